# baseline (speedup 1.0000x reference)
import functools

import jax
import jax.numpy as jnp
from jax import lax
from jax.experimental import pallas as pl
from jax.experimental.pallas import tpu as pltpu
from jax.experimental.pallas import tpu_sc as plsc

B = 16384
F = 26
E = 16
V = 1000
D0 = F * E
EPS = 1e-5
TB = 4096
NT = B // TB
TBM = 8192
NTM = B // TBM


def _sc_gather(table, flat_idx):
    info = plsc.get_sparse_core_info()
    nw = info.num_cores * info.num_subcores
    rows = flat_idx.shape[0]
    rpw = rows // nw
    nch = 8
    ch = rpw // nch

    mesh = plsc.VectorSubcoreMesh(core_axis_name="c", subcore_axis_name="s")

    @functools.partial(
        pl.kernel,
        mesh=mesh,
        out_type=jax.ShapeDtypeStruct((rows, E), jnp.float32),
        scratch_types=[
            pltpu.VMEM_SHARED((F * V, E), jnp.float32),
            pltpu.VMEM((rpw,), jnp.int32),
            pltpu.VMEM((2, ch, E), jnp.float32),
            pltpu.SemaphoreType.DMA,
            pltpu.SemaphoreType.DMA,
        ],
        compiler_params=pltpu.CompilerParams(use_tc_tiling_on_sc=False),
    )
    def gather_kernel(table_hbm, idx_hbm, out_hbm, tbl_s, idx_v, buf, sem0, sem1):
        sid = lax.axis_index("s")
        wid = sid * info.num_cores + lax.axis_index("c")
        base = wid * rpw

        @pl.when(sid == 0)
        def _():
            pltpu.sync_copy(table_hbm, tbl_s)

        pltpu.sync_copy(idx_hbm.at[pl.ds(base, rpw)], idx_v)
        plsc.subcore_barrier()
        sems = (sem0, sem1)

        def start(i):
            return pltpu.async_copy(
                tbl_s.at[idx_v.at[pl.ds(i * ch, ch)]],
                buf.at[i % 2],
                sems[i % 2],
            )

        pending = start(0)
        for i in range(nch):
            nxt = start(i + 1) if i + 1 < nch else None
            pending.wait()
            pltpu.sync_copy(buf.at[i % 2], out_hbm.at[pl.ds(base + i * ch, ch)])
            pending = nxt

    return gather_kernel(table, flat_idx)


def _mlp_body(x0_hbm, W0, W256, Wmid, W128, Wout, gbe256, gbe128, bout,
              out_ref, Xs, Ys, x0buf, sems):
    f32 = jnp.float32

    def finalize(s, q, g, be):
        mu = s / B
        var = q / B - mu * mu
        sc = g * lax.rsqrt(var + EPS)
        return sc, be - mu * sc

    def x0_copy(t, slot):
        return pltpu.make_async_copy(
            x0_hbm.at[pl.ds(t * TB, TB)], x0buf.at[slot], sems.at[slot])

    x0_copy(0, 0).start()

    def body1(t, carry):
        s, q = carry
        slot = lax.rem(t, 2)

        @pl.when(t + 1 < NT)
        def _():
            x0_copy(t + 1, lax.rem(t + 1, 2)).start()

        x0_copy(t, slot).wait()
        f = x0buf[slot].astype(jnp.bfloat16)
        z = jnp.dot(f, W0[:], preferred_element_type=f32)
        Ys[pl.ds(t * TB, TB), :] = z
        return (s + jnp.sum(z, axis=0, keepdims=True),
                q + jnp.sum(z * z, axis=0, keepdims=True))

    s, q = lax.fori_loop(0, NT, body1,
                         (jnp.zeros((1, 256), f32), jnp.zeros((1, 256), f32)))
    scsh = finalize(s, q, gbe256[0:1], gbe256[1:2])

    def sweep(Wmat, win, wout, scsh, res, save, gbe, grow, tbm=TBM):
        sc, sh = scsh

        def body(t, carry):
            s, q = carry
            rows = pl.ds(t * tbm, tbm)
            y = Ys[rows, pl.ds(0, win)]
            a = y * sc + sh
            if res:
                a = a + Xs[rows, pl.ds(0, win)]
            f = jnp.maximum(a, 0.0)
            if save:
                Xs[rows, pl.ds(0, win)] = f
            z = jnp.dot(f.astype(jnp.bfloat16), Wmat, preferred_element_type=f32)
            Ys[rows, pl.ds(0, wout)] = z
            return (s + jnp.sum(z, axis=0, keepdims=True),
                    q + jnp.sum(z * z, axis=0, keepdims=True))

        s, q = lax.fori_loop(0, B // tbm, body,
                             (jnp.zeros((1, wout), f32), jnp.zeros((1, wout), f32)))
        return finalize(s, q, gbe[grow:grow + 1], gbe[grow + 1:grow + 2])

    scsh = sweep(W256[0], 256, 256, scsh, res=False, save=True, gbe=gbe256, grow=2, tbm=B)
    scsh = sweep(W256[1], 256, 256, scsh, res=False, save=False, gbe=gbe256, grow=4, tbm=B)
    scsh = sweep(W256[2], 256, 256, scsh, res=True, save=True, gbe=gbe256, grow=6, tbm=B)
    scsh = sweep(W256[3], 256, 256, scsh, res=False, save=False, gbe=gbe256, grow=8, tbm=B)
    scsh = sweep(Wmid[:], 256, 128, scsh, res=True, save=False, gbe=gbe128, grow=0)
    scsh = sweep(W128[0], 128, 128, scsh, res=False, save=True, gbe=gbe128, grow=2, tbm=B)
    scsh = sweep(W128[1], 128, 128, scsh, res=False, save=False, gbe=gbe128, grow=4, tbm=B)
    scsh = sweep(W128[2], 128, 128, scsh, res=True, save=True, gbe=gbe128, grow=6, tbm=B)
    scsh = sweep(W128[3], 128, 128, scsh, res=False, save=False, gbe=gbe128, grow=8, tbm=B)

    sc, sh = scsh

    y = Ys[:, pl.ds(0, 128)]
    f = y * sc + sh + Xs[:, pl.ds(0, 128)]
    f = jnp.maximum(f, 0.0)
    z = jnp.dot(f.astype(jnp.bfloat16), Wout[:],
                preferred_element_type=f32) + bout[0, 0]
    out_ref[:, :] = 1.0 / (1.0 + jnp.exp(-z))


def _mlp_tc(x0, W0, W256, Wmid, W128, Wout, gbe256, gbe128, bout):
    return pl.pallas_call(
        _mlp_body,
        out_shape=jax.ShapeDtypeStruct((B, 1), jnp.float32),
        in_specs=[pl.BlockSpec(memory_space=pl.ANY)]
                 + [pl.BlockSpec(memory_space=pltpu.VMEM)] * 8,
        out_specs=pl.BlockSpec(memory_space=pltpu.VMEM),
        scratch_shapes=[
            pltpu.VMEM((B, 256), jnp.float32),
            pltpu.VMEM((B, 256), jnp.float32),
            pltpu.VMEM((2, TB, D0), jnp.float32),
            pltpu.SemaphoreType.DMA((2,)),
        ],
        compiler_params=pltpu.CompilerParams(
            vmem_limit_bytes=100 * 1024 * 1024,
        ),
    )(x0, W0, W256, Wmid, W128, Wout, gbe256, gbe128, bout)


def kernel(feature_indices, emb, params):
    table = emb.reshape(F * V, E)
    offs = (jnp.arange(F, dtype=jnp.int32) * V)[None, :]
    flat_idx = (feature_indices + offs).reshape(-1)
    x0 = _sc_gather(table, flat_idx).reshape(B, D0)

    b0, b1 = params['blocks']
    bf16 = jnp.bfloat16
    W0 = b0['W'].astype(bf16)
    W256 = jnp.stack([b0['res'][0]['W1'], b0['res'][0]['W2'],
                      b0['res'][1]['W1'], b0['res'][1]['W2']]).astype(bf16)
    Wmid = b1['W'].astype(bf16)
    W128 = jnp.stack([b1['res'][0]['W1'], b1['res'][0]['W2'],
                      b1['res'][1]['W1'], b1['res'][1]['W2']]).astype(bf16)
    gbe256 = jnp.stack([
        b0['g'], b0['be'],
        b0['res'][0]['g1'], b0['res'][0]['be1'],
        b0['res'][0]['g2'], b0['res'][0]['be2'],
        b0['res'][1]['g1'], b0['res'][1]['be1'],
        b0['res'][1]['g2'], b0['res'][1]['be2'],
    ])
    gbe128 = jnp.stack([
        b1['g'], b1['be'],
        b1['res'][0]['g1'], b1['res'][0]['be1'],
        b1['res'][0]['g2'], b1['res'][0]['be2'],
        b1['res'][1]['g1'], b1['res'][1]['be1'],
        b1['res'][1]['g2'], b1['res'][1]['be2'],
    ])
    bout = params['bout'].reshape(1, 1)

    return _mlp_tc(x0, W0, W256, Wmid, W128, params['Wout'].astype(bf16),
                   gbe256, gbe128, bout)


# speedup vs baseline: 1.0161x; 1.0161x over previous
import functools

import jax
import jax.numpy as jnp
from jax import lax
from jax.experimental import pallas as pl
from jax.experimental.pallas import tpu as pltpu
from jax.experimental.pallas import tpu_sc as plsc

B = 16384
F = 26
E = 16
V = 1000
D0 = F * E
EPS = 1e-5
TB = 4096
NT = B // TB
TBM = 8192
NTM = B // TBM


def _sc_gather(table, flat_idx):
    info = plsc.get_sparse_core_info()
    nw = info.num_cores * info.num_subcores
    rows = flat_idx.shape[0]
    rpw = rows // nw
    nch = 16
    ch = rpw // nch

    mesh = plsc.VectorSubcoreMesh(core_axis_name="c", subcore_axis_name="s")

    @functools.partial(
        pl.kernel,
        mesh=mesh,
        out_type=jax.ShapeDtypeStruct((rows, E), jnp.float32),
        scratch_types=[
            pltpu.VMEM_SHARED((F * V, E), jnp.float32),
            pltpu.VMEM((rpw,), jnp.int32),
            pltpu.VMEM((2, ch, E), jnp.float32),
            pltpu.SemaphoreType.DMA,
            pltpu.SemaphoreType.DMA,
        ],
        compiler_params=pltpu.CompilerParams(use_tc_tiling_on_sc=False),
    )
    def gather_kernel(table_hbm, idx_hbm, out_hbm, tbl_s, idx_v, buf, sem0, sem1):
        sid = lax.axis_index("s")
        wid = sid * info.num_cores + lax.axis_index("c")
        base = wid * rpw

        @pl.when(sid == 0)
        def _():
            pltpu.sync_copy(table_hbm, tbl_s)

        pltpu.sync_copy(idx_hbm.at[pl.ds(base, rpw)], idx_v)
        plsc.subcore_barrier()
        sems = (sem0, sem1)

        def start(i):
            return pltpu.async_copy(
                tbl_s.at[idx_v.at[pl.ds(i * ch, ch)]],
                buf.at[i % 2],
                sems[i % 2],
            )

        pending = start(0)
        for i in range(nch):
            nxt = start(i + 1) if i + 1 < nch else None
            pending.wait()
            pltpu.sync_copy(buf.at[i % 2], out_hbm.at[pl.ds(base + i * ch, ch)])
            pending = nxt

    return gather_kernel(table, flat_idx)


def _mlp_body(x0_hbm, W0, W256, Wmid, W128, Wout, gbe256, gbe128, bout,
              out_ref, Xs, Ys, x0buf, sems):
    f32 = jnp.float32

    def finalize(s, q, g, be):
        mu = s / B
        var = q / B - mu * mu
        sc = g * lax.rsqrt(var + EPS)
        return sc, be - mu * sc

    def x0_copy(t, slot):
        return pltpu.make_async_copy(
            x0_hbm.at[pl.ds(t * TB, TB)], x0buf.at[slot], sems.at[slot])

    x0_copy(0, 0).start()

    def body1(t, carry):
        s, q = carry
        slot = lax.rem(t, 2)

        @pl.when(t + 1 < NT)
        def _():
            x0_copy(t + 1, lax.rem(t + 1, 2)).start()

        x0_copy(t, slot).wait()
        f = x0buf[slot].astype(jnp.bfloat16)
        z = jnp.dot(f, W0[:], preferred_element_type=f32)
        Ys[pl.ds(t * TB, TB), :] = z
        return (s + jnp.sum(z, axis=0, keepdims=True),
                q + jnp.sum(z * z, axis=0, keepdims=True))

    s, q = lax.fori_loop(0, NT, body1,
                         (jnp.zeros((1, 256), f32), jnp.zeros((1, 256), f32)))
    scsh = finalize(s, q, gbe256[0:1], gbe256[1:2])

    def sweep(Wmat, win, wout, scsh, res, save, gbe, grow, tbm=TBM):
        sc, sh = scsh

        def body(t, carry):
            s, q = carry
            rows = pl.ds(t * tbm, tbm)
            y = Ys[rows, pl.ds(0, win)]
            a = y * sc + sh
            if res:
                a = a + Xs[rows, pl.ds(0, win)]
            f = jnp.maximum(a, 0.0)
            if save:
                Xs[rows, pl.ds(0, win)] = f
            z = jnp.dot(f.astype(jnp.bfloat16), Wmat, preferred_element_type=f32)
            Ys[rows, pl.ds(0, wout)] = z
            return (s + jnp.sum(z, axis=0, keepdims=True),
                    q + jnp.sum(z * z, axis=0, keepdims=True))

        s, q = lax.fori_loop(0, B // tbm, body,
                             (jnp.zeros((1, wout), f32), jnp.zeros((1, wout), f32)))
        return finalize(s, q, gbe[grow:grow + 1], gbe[grow + 1:grow + 2])

    scsh = sweep(W256[0], 256, 256, scsh, res=False, save=True, gbe=gbe256, grow=2)
    scsh = sweep(W256[1], 256, 256, scsh, res=False, save=False, gbe=gbe256, grow=4)
    scsh = sweep(W256[2], 256, 256, scsh, res=True, save=True, gbe=gbe256, grow=6)
    scsh = sweep(W256[3], 256, 256, scsh, res=False, save=False, gbe=gbe256, grow=8)
    scsh = sweep(Wmid[:], 256, 128, scsh, res=True, save=False, gbe=gbe128, grow=0)
    scsh = sweep(W128[0], 128, 128, scsh, res=False, save=True, gbe=gbe128, grow=2, tbm=B)
    scsh = sweep(W128[1], 128, 128, scsh, res=False, save=False, gbe=gbe128, grow=4, tbm=B)
    scsh = sweep(W128[2], 128, 128, scsh, res=True, save=True, gbe=gbe128, grow=6, tbm=B)
    scsh = sweep(W128[3], 128, 128, scsh, res=False, save=False, gbe=gbe128, grow=8, tbm=B)

    sc, sh = scsh

    y = Ys[:, pl.ds(0, 128)]
    f = y * sc + sh + Xs[:, pl.ds(0, 128)]
    f = jnp.maximum(f, 0.0)
    z = jnp.dot(f.astype(jnp.bfloat16), Wout[:],
                preferred_element_type=f32) + bout[0, 0]
    out_ref[:, :] = 1.0 / (1.0 + jnp.exp(-z))


def _mlp_tc(x0, W0, W256, Wmid, W128, Wout, gbe256, gbe128, bout):
    return pl.pallas_call(
        _mlp_body,
        out_shape=jax.ShapeDtypeStruct((B, 1), jnp.float32),
        in_specs=[pl.BlockSpec(memory_space=pl.ANY)]
                 + [pl.BlockSpec(memory_space=pltpu.VMEM)] * 8,
        out_specs=pl.BlockSpec(memory_space=pltpu.VMEM),
        scratch_shapes=[
            pltpu.VMEM((B, 256), jnp.float32),
            pltpu.VMEM((B, 256), jnp.float32),
            pltpu.VMEM((2, TB, D0), jnp.float32),
            pltpu.SemaphoreType.DMA((2,)),
        ],
        compiler_params=pltpu.CompilerParams(
            vmem_limit_bytes=100 * 1024 * 1024,
        ),
    )(x0, W0, W256, Wmid, W128, Wout, gbe256, gbe128, bout)


def kernel(feature_indices, emb, params):
    table = emb.reshape(F * V, E)
    offs = (jnp.arange(F, dtype=jnp.int32) * V)[None, :]
    flat_idx = (feature_indices + offs).reshape(-1)
    x0 = _sc_gather(table, flat_idx).reshape(B, D0)

    b0, b1 = params['blocks']
    bf16 = jnp.bfloat16
    W0 = b0['W'].astype(bf16)
    W256 = jnp.stack([b0['res'][0]['W1'], b0['res'][0]['W2'],
                      b0['res'][1]['W1'], b0['res'][1]['W2']]).astype(bf16)
    Wmid = b1['W'].astype(bf16)
    W128 = jnp.stack([b1['res'][0]['W1'], b1['res'][0]['W2'],
                      b1['res'][1]['W1'], b1['res'][1]['W2']]).astype(bf16)
    gbe256 = jnp.stack([
        b0['g'], b0['be'],
        b0['res'][0]['g1'], b0['res'][0]['be1'],
        b0['res'][0]['g2'], b0['res'][0]['be2'],
        b0['res'][1]['g1'], b0['res'][1]['be1'],
        b0['res'][1]['g2'], b0['res'][1]['be2'],
    ])
    gbe128 = jnp.stack([
        b1['g'], b1['be'],
        b1['res'][0]['g1'], b1['res'][0]['be1'],
        b1['res'][0]['g2'], b1['res'][0]['be2'],
        b1['res'][1]['g1'], b1['res'][1]['be1'],
        b1['res'][1]['g2'], b1['res'][1]['be2'],
    ])
    bout = params['bout'].reshape(1, 1)

    return _mlp_tc(x0, W0, W256, Wmid, W128, params['Wout'].astype(bf16),
                   gbe256, gbe128, bout)
